# baseline (speedup 1.0000x reference)
import jax
import jax.numpy as jnp
from jax import lax
from jax.experimental import pallas as pl
from jax.experimental.pallas import tpu as pltpu
from jax.experimental.pallas import tpu_sc as plsc

EMB = 64
NC, NS = 2, 16
NW = NC * NS


def _emb_body(x_hbm, table_hbm, out_hbm, idx_all, rows_v, gsem, ssem):
    n_rows, n_cols = x_hbm.shape
    rows_w = n_rows // NW
    wid = lax.axis_index("s") * NC + lax.axis_index("c")
    base = wid * rows_w

    pltpu.sync_copy(x_hbm.at[pl.ds(base, rows_w)], idx_all)

    def gather_and_store(i, s):
        pltpu.async_copy(
            table_hbm.at[idx_all.at[i]], rows_v.at[s], gsem
        ).wait()
        pltpu.async_copy(rows_v.at[s], out_hbm.at[base + i], ssem)

    def drain_one_store(s):
        pltpu.make_async_copy(rows_v.at[s], out_hbm.at[base], ssem).wait()

    gather_and_store(0, 0)
    gather_and_store(1, 1)

    @pl.loop(2, rows_w, step=2)
    def _pair(i):
        for s in range(2):
            drain_one_store(s)
            gather_and_store(i + s, s)

    drain_one_store(0)
    drain_one_store(1)


def kernel(x, W):
    n_rows, n_cols = x.shape
    xi = x.astype(jnp.int32)
    mesh = plsc.VectorSubcoreMesh(
        core_axis_name="c", subcore_axis_name="s",
        num_cores=NC, num_subcores=NS,
    )
    return pl.kernel(
        _emb_body,
        out_type=jax.ShapeDtypeStruct((n_rows, n_cols, EMB), jnp.float32),
        mesh=mesh,
        scratch_types=[
            pltpu.VMEM((n_rows // NW, n_cols), jnp.int32),
            pltpu.VMEM((2, n_cols, EMB), jnp.float32),
            pltpu.SemaphoreType.DMA,
            pltpu.SemaphoreType.DMA,
        ],
        compiler_params=pltpu.CompilerParams(use_tc_tiling_on_sc=False),
    )(xi, W)
